# Initial kernel scaffold; baseline (speedup 1.0000x reference)
import jax
import jax.numpy as jnp
from jax.experimental import pallas as pl


def kernel(vgg_end, interm, branchA_end):
    raise NotImplementedError("write your pallas kernel here")


# speedup vs baseline: 1.0610x; 1.0610x over previous
import functools

import jax
import jax.numpy as jnp
from jax import lax
from jax.experimental import pallas as pl
from jax.experimental.pallas import tpu as pltpu
from jax.experimental.pallas import tpu_sc as plsc

THRESH = 0.5
B = 256
HW = 49
C = 512
K = 1000

NC = 2
NS = 16
NW = NC * NS
BPW = B // NW

APAD = 64
LANES = 16


def _sc_argmax_gather(branchA, intermf):
    mesh = plsc.VectorSubcoreMesh(
        core_axis_name="c", subcore_axis_name="s",
        num_cores=NC, num_subcores=NS)

    @functools.partial(
        pl.kernel,
        out_type=jax.ShapeDtypeStruct((B * APAD,), jnp.float32),
        mesh=mesh,
        compiler_params=pltpu.CompilerParams(
            needs_layout_passes=False, use_tc_tiling_on_sc=False),
        scratch_types=[
            pltpu.VMEM((BPW, K), jnp.float32),
            pltpu.VMEM((BPW, APAD), jnp.int32),
            pltpu.VMEM((BPW * APAD,), jnp.float32),
            pltpu.VMEM((LANES,), jnp.float32),
            pltpu.VMEM((LANES,), jnp.int32),
            pltpu.SemaphoreType.DMA,
        ],
    )
    def sc_k(branchA_hbm, intermf_hbm, out_hbm, rowbuf, idxbuf, valbuf,
             tmpv, tmpi, sem):
        wid = lax.axis_index("s") * NC + lax.axis_index("c")
        base = wid * BPW
        pltpu.sync_copy(branchA_hbm.at[pl.ds(base, BPW), :], rowbuf)
        lanes = lax.iota(jnp.int32, LANES)
        copies = []
        for s in range(BPW):
            def body(c, carry):
                bv, bi = carry
                v = rowbuf[s, pl.ds(c * LANES, LANES)]
                g = c * LANES + lanes
                take = v > bv
                return jnp.where(take, v, bv), jnp.where(take, g, bi)

            init = (jnp.full((LANES,), -jnp.inf, jnp.float32),
                    jnp.zeros((LANES,), jnp.int32))
            bv, bi = lax.fori_loop(0, K // LANES, body, init)
            v = rowbuf[s, pl.ds(K - LANES, LANES)]
            g = (K - LANES) + lanes
            take = v > bv
            bv = jnp.where(take, v, bv)
            bi = jnp.where(take, g, bi)
            for sh in (8, 4, 2, 1):
                tmpv[...] = bv
                tmpi[...] = bi
                perm = jnp.bitwise_xor(lanes, sh)
                pv = plsc.load_gather(tmpv, [perm])
                pi = plsc.load_gather(tmpi, [perm])
                better = (pv > bv) | ((pv == bv) & (pi < bi))
                bv = jnp.where(better, pv, bv)
                bi = jnp.where(better, pi, bi)
            fl = (base + s) * (HW * K) + bi
            for j in range(APAD // LANES):
                p = jnp.minimum(j * LANES + lanes, HW - 1)
                idxbuf[s, pl.ds(j * LANES, LANES)] = fl + p * K
            copies.append(pltpu.async_copy(
                intermf_hbm.at[idxbuf.at[s]],
                valbuf.at[pl.ds(s * APAD, APAD)], sem))
        for c in copies:
            c.wait()
        pltpu.sync_copy(valbuf, out_hbm.at[pl.ds(base * APAD, BPW * APAD)])

    return sc_k(branchA, intermf)


def _tc_apply(vgg3, a3):
    BC = 8

    def body(vgg_ref, a_ref, out_ref):
        a = a_ref[...]
        m = jnp.where(a > THRESH, a, jnp.zeros_like(a))
        out_ref[...] = vgg_ref[...] - m

    return pl.pallas_call(
        body,
        grid=(B // BC,),
        in_specs=[
            pl.BlockSpec((BC, HW, C), lambda i: (i, 0, 0)),
            pl.BlockSpec((BC, HW, 1), lambda i: (i, 0, 0)),
        ],
        out_specs=pl.BlockSpec((BC, HW, C), lambda i: (i, 0, 0)),
        out_shape=jax.ShapeDtypeStruct((B, HW, C), jnp.float32),
    )(vgg3, a3)


def kernel(vgg_end, interm, branchA_end):
    intermf = interm.reshape(B * HW * K)
    vgg3 = vgg_end.reshape(B, HW, C)
    a = _sc_argmax_gather(branchA_end, intermf)
    a = a.reshape(B, APAD)[:, :HW]
    out = _tc_apply(vgg3, a.reshape(B, HW, 1))
    return out.reshape(vgg_end.shape)
